# baseline (speedup 1.0000x reference)
import functools

import jax
import jax.numpy as jnp
from jax import lax
from jax.experimental import pallas as pl
from jax.experimental.pallas import tpu as pltpu
from jax.experimental.pallas import tpu_sc as plsc

_GROUPS = 2
_NUM_VARS = 512
_VAR_DIM = 64
_OUT_DIM = _GROUPS * _VAR_DIM
_N_TILE = 256
_LANES = 16


def _logits_kernel(n_tokens, x_ref, w_ref, b_ref, cb_ref,
                   idx_ref, probs_ref, cbp_ref):
    i = pl.program_id(0)
    logits = jax.lax.dot_general(
        x_ref[:], w_ref[:],
        dimension_numbers=(((1,), (1,)), ((), ())),
        preferred_element_type=jnp.float32,
    ) + b_ref[:]
    inv_n = jnp.float32(1.0 / n_tokens)
    psums = []
    ks = []
    for g in range(_GROUPS):
        lg = logits[:, g * _NUM_VARS:(g + 1) * _NUM_VARS]
        m = jnp.max(lg, axis=-1, keepdims=True)
        e = jnp.exp(lg - m)
        s = jnp.sum(e, axis=-1, keepdims=True)
        psums.append(jnp.sum(e / s, axis=0) * inv_n)
        idx = jax.lax.broadcasted_iota(jnp.int32, lg.shape, 1)
        k = jnp.min(jnp.where(lg == m, idx, _NUM_VARS), axis=-1, keepdims=True)
        ks.append(k + g * _NUM_VARS)
    idx_ref[:] = jnp.concatenate(ks, axis=1)
    psum = jnp.concatenate(psums).reshape(1, _GROUPS * _NUM_VARS)

    @pl.when(i == 0)
    def _():
        probs_ref[:] = psum
        z = jnp.zeros((_NUM_VARS, _VAR_DIM), jnp.float32)
        top = jnp.concatenate([cb_ref[:_NUM_VARS, :], z], axis=1)
        bot = jnp.concatenate([z, cb_ref[_NUM_VARS:, :]], axis=1)
        cbp_ref[:] = jnp.concatenate([top, bot], axis=0)

    @pl.when(i != 0)
    def _():
        probs_ref[:] = probs_ref[:] + psum


def _make_gather(n_tokens):
    info = plsc.get_sparse_core_info()
    nc, ns = info.num_cores, info.num_subcores
    nw = nc * ns
    tpw = n_tokens // nw
    mesh = plsc.VectorSubcoreMesh(core_axis_name="c", subcore_axis_name="s")

    @functools.partial(
        pl.kernel, mesh=mesh,
        out_type=jax.ShapeDtypeStruct((n_tokens, _OUT_DIM), jnp.float32),
        scratch_types=[
            pltpu.VMEM((_GROUPS * tpw,), jnp.int32),
            pltpu.VMEM((_GROUPS * tpw, _OUT_DIM), jnp.float32),
            pltpu.VMEM((tpw, _OUT_DIM), jnp.float32),
            pltpu.SemaphoreType.DMA,
        ],
    )
    def gather(cb_hbm, idx_hbm, out_hbm, idx_v, rows_v, out_v, sem):
        wid = lax.axis_index("s") * nc + lax.axis_index("c")
        base = wid * tpw
        pltpu.sync_copy(idx_hbm.at[pl.ds(_GROUPS * base, _GROUPS * tpw)], idx_v)
        pltpu.async_copy(cb_hbm.at[idx_v], rows_v, sem).wait()

        def row(i, _):
            for j in range(_OUT_DIM // _LANES):
                s = pl.ds(j * _LANES, _LANES)
                out_v[i, s] = rows_v[2 * i, s] + rows_v[2 * i + 1, s]
            return 0

        lax.fori_loop(0, tpw, row, 0)
        pltpu.sync_copy(out_v, out_hbm.at[pl.ds(base, tpw)])

    return gather


def kernel(x, W, b, codebook):
    bsz, t, d = x.shape
    n = bsz * t
    flat = x.reshape(n, d)
    cb = codebook.reshape(_GROUPS * _NUM_VARS, _VAR_DIM)
    grid = n // _N_TILE
    idx, probs, cb_pad = pl.pallas_call(
        functools.partial(_logits_kernel, n),
        grid=(grid,),
        in_specs=[
            pl.BlockSpec((_N_TILE, d), lambda i: (i, 0)),
            pl.BlockSpec((_GROUPS * _NUM_VARS, d), lambda i: (0, 0)),
            pl.BlockSpec((1, _GROUPS * _NUM_VARS), lambda i: (0, 0)),
            pl.BlockSpec((_GROUPS * _NUM_VARS, _VAR_DIM), lambda i: (0, 0)),
        ],
        out_specs=[
            pl.BlockSpec((_N_TILE, _GROUPS), lambda i: (i, 0)),
            pl.BlockSpec((1, _GROUPS * _NUM_VARS), lambda i: (0, 0)),
            pl.BlockSpec((_GROUPS * _NUM_VARS, _OUT_DIM), lambda i: (0, 0)),
        ],
        out_shape=[
            jax.ShapeDtypeStruct((n, _GROUPS), jnp.int32),
            jax.ShapeDtypeStruct((1, _GROUPS * _NUM_VARS), jnp.float32),
            jax.ShapeDtypeStruct((_GROUPS * _NUM_VARS, _OUT_DIM), jnp.float32),
        ],
    )(flat, W, b.reshape(1, -1), cb)
    out = _make_gather(n)(cb_pad, idx.reshape(n * _GROUPS))
    avg_probs = probs.reshape(_GROUPS, _NUM_VARS)
    return out.reshape(bsz, t, _OUT_DIM), avg_probs


# speedup vs baseline: 1.0117x; 1.0117x over previous
import functools

import jax
import jax.numpy as jnp
from jax import lax
from jax.experimental import pallas as pl
from jax.experimental.pallas import tpu as pltpu
from jax.experimental.pallas import tpu_sc as plsc

_GROUPS = 2
_NUM_VARS = 512
_VAR_DIM = 64
_OUT_DIM = _GROUPS * _VAR_DIM
_N_TILE = 256
_LANES = 16


def _logits_kernel(n_tokens, x_ref, w_ref, b_ref, idx_ref, probs_ref):
    i = pl.program_id(0)
    logits = jax.lax.dot_general(
        x_ref[:], w_ref[:],
        dimension_numbers=(((1,), (1,)), ((), ())),
        preferred_element_type=jnp.float32,
    ) + b_ref[:]
    inv_n = jnp.float32(1.0 / n_tokens)
    psums = []
    ks = []
    for g in range(_GROUPS):
        lg = logits[:, g * _NUM_VARS:(g + 1) * _NUM_VARS]
        m = jnp.max(lg, axis=-1, keepdims=True)
        e = jnp.exp(lg - m)
        s = jnp.sum(e, axis=-1, keepdims=True)
        psums.append(jnp.sum(e / s, axis=0) * inv_n)
        idx = jax.lax.broadcasted_iota(jnp.int32, lg.shape, 1)
        k = jnp.min(jnp.where(lg == m, idx, _NUM_VARS), axis=-1, keepdims=True)
        ks.append(k + g * _NUM_VARS)
    idx_ref[:] = jnp.concatenate(ks, axis=1)
    psum = jnp.concatenate(psums).reshape(1, _GROUPS * _NUM_VARS)

    @pl.when(i == 0)
    def _():
        probs_ref[:] = psum

    @pl.when(i != 0)
    def _():
        probs_ref[:] = probs_ref[:] + psum


def _make_gather(n_tokens):
    info = plsc.get_sparse_core_info()
    nc, ns = info.num_cores, info.num_subcores
    nw = nc * ns
    tpw = n_tokens // nw
    mesh = plsc.VectorSubcoreMesh(core_axis_name="c", subcore_axis_name="s")

    @functools.partial(
        pl.kernel, mesh=mesh,
        out_type=jax.ShapeDtypeStruct((n_tokens, _OUT_DIM), jnp.float32),
        scratch_types=[
            pltpu.VMEM((_GROUPS * tpw,), jnp.int32),
            pltpu.VMEM((_GROUPS * tpw, _OUT_DIM), jnp.float32),
            pltpu.VMEM((tpw, _OUT_DIM), jnp.float32),
            pltpu.SemaphoreType.DMA,
        ],
    )
    def gather(cb_hbm, idx_hbm, out_hbm, idx_v, rows_v, out_v, sem):
        wid = lax.axis_index("s") * nc + lax.axis_index("c")
        base = wid * tpw
        pltpu.sync_copy(idx_hbm.at[pl.ds(_GROUPS * base, _GROUPS * tpw)], idx_v)
        pltpu.async_copy(cb_hbm.at[idx_v], rows_v, sem).wait()

        def row(i, _):
            for j in range(_OUT_DIM // _LANES):
                s = pl.ds(j * _LANES, _LANES)
                out_v[i, s] = rows_v[2 * i, s] + rows_v[2 * i + 1, s]
            return 0

        lax.fori_loop(0, tpw, row, 0)
        pltpu.sync_copy(out_v, out_hbm.at[pl.ds(base, tpw)])

    return gather


def kernel(x, W, b, codebook):
    bsz, t, d = x.shape
    n = bsz * t
    flat = x.reshape(n, d)
    cb = codebook.reshape(_GROUPS * _NUM_VARS, _VAR_DIM)
    cb_pad = jnp.concatenate(
        [jnp.pad(cb[:_NUM_VARS], ((0, 0), (0, _VAR_DIM))),
         jnp.pad(cb[_NUM_VARS:], ((0, 0), (_VAR_DIM, 0)))], axis=0)
    grid = n // _N_TILE
    idx, probs = pl.pallas_call(
        functools.partial(_logits_kernel, n),
        grid=(grid,),
        in_specs=[
            pl.BlockSpec((_N_TILE, d), lambda i: (i, 0)),
            pl.BlockSpec((_GROUPS * _NUM_VARS, d), lambda i: (0, 0)),
            pl.BlockSpec((1, _GROUPS * _NUM_VARS), lambda i: (0, 0)),
        ],
        out_specs=[
            pl.BlockSpec((_N_TILE, _GROUPS), lambda i: (i, 0)),
            pl.BlockSpec((1, _GROUPS * _NUM_VARS), lambda i: (0, 0)),
        ],
        out_shape=[
            jax.ShapeDtypeStruct((n, _GROUPS), jnp.int32),
            jax.ShapeDtypeStruct((1, _GROUPS * _NUM_VARS), jnp.float32),
        ],
    )(flat, W, b.reshape(1, -1))
    out = _make_gather(n)(cb_pad, idx.reshape(n * _GROUPS))
    avg_probs = probs.reshape(_GROUPS, _NUM_VARS)
    return out.reshape(bsz, t, _OUT_DIM), avg_probs


# speedup vs baseline: 1.0730x; 1.0606x over previous
import functools

import jax
import jax.numpy as jnp
from jax import lax
from jax.experimental import pallas as pl
from jax.experimental.pallas import tpu as pltpu
from jax.experimental.pallas import tpu_sc as plsc

_GROUPS = 2
_NUM_VARS = 512
_VAR_DIM = 64
_OUT_DIM = _GROUPS * _VAR_DIM
_N_TILE = 256
_LANES = 16


def _logits_kernel(n_tokens, x_ref, w_ref, b_ref, idx_ref, probs_ref):
    i = pl.program_id(0)
    logits = jax.lax.dot_general(
        x_ref[:], w_ref[:],
        dimension_numbers=(((1,), (1,)), ((), ())),
        preferred_element_type=jnp.float32,
    ) + b_ref[:]
    inv_n = jnp.float32(1.0 / n_tokens)
    psums = []
    ks = []
    for g in range(_GROUPS):
        lg = logits[:, g * _NUM_VARS:(g + 1) * _NUM_VARS]
        m = jnp.max(lg, axis=-1, keepdims=True)
        e = jnp.exp(lg - m)
        s = jnp.sum(e, axis=-1, keepdims=True)
        psums.append(jnp.sum(e / s, axis=0) * inv_n)
        idx = jax.lax.broadcasted_iota(jnp.int32, lg.shape, 1)
        k = jnp.min(jnp.where(lg == m, idx, _NUM_VARS), axis=-1, keepdims=True)
        ks.append(k + g * _NUM_VARS)
    idx_ref[:] = jnp.concatenate(ks, axis=1)
    psum = jnp.concatenate(psums).reshape(1, _GROUPS * _NUM_VARS)

    @pl.when(i == 0)
    def _():
        probs_ref[:] = psum

    @pl.when(i != 0)
    def _():
        probs_ref[:] = probs_ref[:] + psum


def _make_gather(n_tokens):
    info = plsc.get_sparse_core_info()
    nc, ns = info.num_cores, info.num_subcores
    nw = nc * ns
    n_rows = n_tokens * _GROUPS
    rpw = n_rows // nw
    mesh = plsc.VectorSubcoreMesh(core_axis_name="c", subcore_axis_name="s")

    @functools.partial(
        pl.kernel, mesh=mesh,
        out_type=jax.ShapeDtypeStruct((n_rows, _VAR_DIM), jnp.float32),
        scratch_types=[
            pltpu.VMEM((rpw,), jnp.int32),
            pltpu.VMEM((rpw, _VAR_DIM), jnp.float32),
            pltpu.SemaphoreType.DMA,
        ],
        compiler_params=pltpu.CompilerParams(use_tc_tiling_on_sc=False),
    )
    def gather(cb_hbm, idx_hbm, out_hbm, idx_v, rows_v, sem):
        wid = lax.axis_index("s") * nc + lax.axis_index("c")
        base = wid * rpw
        pltpu.sync_copy(idx_hbm.at[pl.ds(base, rpw)], idx_v)
        pltpu.async_copy(cb_hbm.at[idx_v], rows_v, sem).wait()
        pltpu.sync_copy(rows_v, out_hbm.at[pl.ds(base, rpw)])

    return gather


def kernel(x, W, b, codebook):
    bsz, t, d = x.shape
    n = bsz * t
    flat = x.reshape(n, d)
    cb = codebook.reshape(_GROUPS * _NUM_VARS, _VAR_DIM)
    grid = n // _N_TILE
    idx, probs = pl.pallas_call(
        functools.partial(_logits_kernel, n),
        grid=(grid,),
        in_specs=[
            pl.BlockSpec((_N_TILE, d), lambda i: (i, 0)),
            pl.BlockSpec((_GROUPS * _NUM_VARS, d), lambda i: (0, 0)),
            pl.BlockSpec((1, _GROUPS * _NUM_VARS), lambda i: (0, 0)),
        ],
        out_specs=[
            pl.BlockSpec((_N_TILE, _GROUPS), lambda i: (i, 0)),
            pl.BlockSpec((1, _GROUPS * _NUM_VARS), lambda i: (0, 0)),
        ],
        out_shape=[
            jax.ShapeDtypeStruct((n, _GROUPS), jnp.int32),
            jax.ShapeDtypeStruct((1, _GROUPS * _NUM_VARS), jnp.float32),
        ],
    )(flat, W, b.reshape(1, -1))
    out = _make_gather(n)(cb, idx.reshape(n * _GROUPS))
    avg_probs = probs.reshape(_GROUPS, _NUM_VARS)
    return out.reshape(bsz, t, _OUT_DIM), avg_probs
